# Initial kernel scaffold; baseline (speedup 1.0000x reference)
import jax
import jax.numpy as jnp
from jax.experimental import pallas as pl


def kernel(feats, edge_index, W1, b1, W2, b2):
    raise NotImplementedError("write your pallas kernel here")


# speedup vs baseline: 20.2571x; 20.2571x over previous
import jax
import jax.numpy as jnp
from jax import lax
from jax.experimental import pallas as pl
from jax.experimental.pallas import tpu as pltpu
from jax.experimental.pallas import tpu_sc as plsc

N = 10000
E = 320000
D_IN = 128
NPAD = 10240
NTILES = 16
EPT = E // NTILES
CHUNK = 2000
NCHUNKS = EPT // CHUNK
RPT = NPAD // NTILES


def _rsqrt16(x):
    i = lax.bitcast_convert_type(x, jnp.int32)
    i = jnp.int32(0x5F3759DF) - lax.shift_right_logical(i, 1)
    y = lax.bitcast_convert_type(i, jnp.float32)
    for _ in range(3):
        y = y * (jnp.float32(1.5) - jnp.float32(0.5) * x * y * y)
    return jnp.where(x > jnp.float32(0.5), y, jnp.float32(0.0))


def _tc_body(x_ref, w1_ref, w2_ref, b1_ref, y_ref, c_ref):
    hi = lax.Precision.HIGHEST
    w12 = jnp.dot(w1_ref[...], w2_ref[...], precision=hi,
                  preferred_element_type=jnp.float32)
    y_ref[...] = jnp.dot(x_ref[...], w12, precision=hi,
                         preferred_element_type=jnp.float32)
    c_ref[...] = jnp.dot(b1_ref[...], w2_ref[...], precision=hi,
                         preferred_element_type=jnp.float32)


_tc_call = pl.pallas_call(
    _tc_body,
    out_shape=[
        jax.ShapeDtypeStruct((NPAD, 2), jnp.float32),
        jax.ShapeDtypeStruct((1, 2), jnp.float32),
    ],
)


def _sc_body(src_hbm, dst_hbm, yc0_hbm, yc1_hbm, cvec_hbm,
             o_hbm,
             dego, degi, ns_t, nd_t, g10, g11, g20, g21,
             a10, a11, a1s, a20, a21,
             ids, idc, m0, m1, m2, ones_v, zb, cb, nsb, ndb, pb, wa, wb):
    wid = lax.axis_index("s")
    r0 = wid * RPT
    e0 = wid * EPT
    rsl = pl.ds(r0, RPT)

    zero16 = jnp.zeros((16,), jnp.float32)
    one16 = jnp.full((16,), 1.0, jnp.float32)

    @pl.loop(0, RPT // 16)
    def _(i):
        zb[pl.ds(i * 16, 16)] = zero16

    @pl.loop(0, CHUNK // 16)
    def _(i):
        ones_v[pl.ds(i * 16, 16)] = one16

    pltpu.sync_copy(cvec_hbm, cb)
    for tbl in (dego, degi, a10, a11, a1s, a20, a21):
        pltpu.sync_copy(zb, tbl.at[rsl])
    plsc.subcore_barrier()

    @pl.loop(0, NCHUNKS)
    def _(k):
        base = e0 + k * CHUNK
        pltpu.sync_copy(src_hbm.at[pl.ds(base, CHUNK)], ids)
        pltpu.sync_copy(dst_hbm.at[pl.ds(base, CHUNK)], idc)
        pltpu.sync_copy(ones_v, dego.at[ids], add=True)
        pltpu.sync_copy(ones_v, degi.at[idc], add=True)
    plsc.subcore_barrier()

    pltpu.sync_copy(dego.at[rsl], wa)
    pltpu.sync_copy(degi.at[rsl], wb)

    @pl.loop(0, RPT // 16)
    def _(i):
        sl = pl.ds(i * 16, 16)
        nsb[sl] = _rsqrt16(wa[sl])
        ndb[sl] = _rsqrt16(wb[sl])

    pltpu.sync_copy(nsb, ns_t.at[rsl])
    pltpu.sync_copy(ndb, nd_t.at[rsl])

    pltpu.sync_copy(yc0_hbm.at[rsl], wa)
    pltpu.sync_copy(yc1_hbm.at[rsl], wb)

    @pl.loop(0, RPT // 16)
    def _(i):
        sl = pl.ds(i * 16, 16)
        wa[sl] = wa[sl] * nsb[sl]
        wb[sl] = wb[sl] * nsb[sl]

    pltpu.sync_copy(wa, g10.at[rsl])
    pltpu.sync_copy(wb, g11.at[rsl])
    plsc.subcore_barrier()

    @pl.loop(0, NCHUNKS)
    def _(k):
        base = e0 + k * CHUNK
        pltpu.sync_copy(src_hbm.at[pl.ds(base, CHUNK)], ids)
        pltpu.sync_copy(dst_hbm.at[pl.ds(base, CHUNK)], idc)
        pltpu.sync_copy(g10.at[ids], m0)
        pltpu.sync_copy(g11.at[ids], m1)
        pltpu.sync_copy(ns_t.at[ids], m2)
        pltpu.sync_copy(m0, a10.at[idc], add=True)
        pltpu.sync_copy(m1, a11.at[idc], add=True)
        pltpu.sync_copy(m2, a1s.at[idc], add=True)
    plsc.subcore_barrier()

    pltpu.sync_copy(a10.at[rsl], wa)
    pltpu.sync_copy(a11.at[rsl], wb)
    pltpu.sync_copy(a1s.at[rsl], pb)

    @pl.loop(0, RPT // 16)
    def _(i):
        sl = pl.ds(i * 16, 16)
        s = nsb[sl] * ndb[sl]
        wa[sl] = wa[sl] * s
        wb[sl] = wb[sl] * s
        pb[sl] = pb[sl] * ndb[sl]

    pltpu.sync_copy(wa, g20.at[rsl])
    pltpu.sync_copy(wb, g21.at[rsl])
    plsc.subcore_barrier()

    @pl.loop(0, NCHUNKS)
    def _(k):
        base = e0 + k * CHUNK
        pltpu.sync_copy(src_hbm.at[pl.ds(base, CHUNK)], ids)
        pltpu.sync_copy(dst_hbm.at[pl.ds(base, CHUNK)], idc)
        pltpu.sync_copy(g20.at[ids], m0)
        pltpu.sync_copy(g21.at[ids], m1)
        pltpu.sync_copy(m0, a20.at[idc], add=True)
        pltpu.sync_copy(m1, a21.at[idc], add=True)
    plsc.subcore_barrier()

    pltpu.sync_copy(a20.at[rsl], wa)
    pltpu.sync_copy(a21.at[rsl], wb)
    c0 = cb[0]
    c1 = cb[1]
    b20 = cb[2]
    b21 = cb[3]

    @pl.loop(0, RPT // 16)
    def _(i):
        sl = pl.ds(i * 16, 16)
        wa[sl] = ndb[sl] * wa[sl] + pb[sl] * c0 + b20
        wb[sl] = ndb[sl] * wb[sl] + pb[sl] * c1 + b21

    pltpu.sync_copy(wa, o_hbm.at[pl.ds(r0, RPT)])
    pltpu.sync_copy(wb, o_hbm.at[pl.ds(NPAD + r0, RPT)])


_sc_call = pl.kernel(
    _sc_body,
    out_type=jax.ShapeDtypeStruct((2 * NPAD,), jnp.float32),
    mesh=plsc.VectorSubcoreMesh(
        core_axis_name="c", subcore_axis_name="s", num_cores=1),
    scratch_types=[pltpu.VMEM_SHARED((NPAD,), jnp.float32)] * 13 + [
        pltpu.VMEM((CHUNK,), jnp.int32),
        pltpu.VMEM((CHUNK,), jnp.int32),
        pltpu.VMEM((CHUNK,), jnp.float32),
        pltpu.VMEM((CHUNK,), jnp.float32),
        pltpu.VMEM((CHUNK,), jnp.float32),
        pltpu.VMEM((CHUNK,), jnp.float32),
        pltpu.VMEM((RPT,), jnp.float32),
        pltpu.VMEM((4, 16), jnp.float32),
        pltpu.VMEM((RPT,), jnp.float32),
        pltpu.VMEM((RPT,), jnp.float32),
        pltpu.VMEM((RPT,), jnp.float32),
        pltpu.VMEM((RPT,), jnp.float32),
        pltpu.VMEM((RPT,), jnp.float32),
    ],
)


def kernel(feats, edge_index, W1, b1, W2, b2):
    xp = jnp.zeros((NPAD, D_IN), jnp.float32).at[:N].set(feats)
    y2, crow = _tc_call(xp, W1, W2, b1.reshape(1, D_IN))
    yc = y2.T
    cvec = (jnp.concatenate([crow[0], b2]).astype(jnp.float32).reshape(4, 1)
            * jnp.ones((1, 16), jnp.float32))
    o_flat = _sc_call(edge_index[0], edge_index[1], yc[0], yc[1], cvec)
    return o_flat.reshape(2, NPAD)[:, :N].T


# speedup vs baseline: 20.8801x; 1.0308x over previous
import jax
import jax.numpy as jnp
from jax import lax
from jax.experimental import pallas as pl
from jax.experimental.pallas import tpu as pltpu
from jax.experimental.pallas import tpu_sc as plsc

N = 10000
E = 320000
D_IN = 128
NPAD = 10240
NCORES = 2
NTILES = 16
NWORK = NCORES * NTILES
EPC = E // NWORK
CHUNK = 2000
NCHUNKS = EPC // CHUNK
RPT = NPAD // NTILES

_MESH = plsc.VectorSubcoreMesh(
    core_axis_name="c", subcore_axis_name="s", num_cores=NCORES)
_SC_PARAMS = pltpu.CompilerParams(use_tc_tiling_on_sc=False)


def _deg_body(src_hbm, dst_hbm, degp_hbm, dego, degi, ids, idc, ones_v, zb):
    cid = lax.axis_index("c")
    wid = lax.axis_index("s")
    r0 = wid * RPT
    rsl = pl.ds(r0, RPT)
    e0 = (cid * NTILES + wid) * EPC

    zero16 = jnp.zeros((16,), jnp.float32)
    one16 = jnp.full((16,), 1.0, jnp.float32)

    @pl.loop(0, RPT // 16)
    def _(i):
        zb[pl.ds(i * 16, 16)] = zero16

    @pl.loop(0, CHUNK // 16)
    def _(i):
        ones_v[pl.ds(i * 16, 16)] = one16

    pltpu.sync_copy(zb, dego.at[rsl])
    pltpu.sync_copy(zb, degi.at[rsl])
    plsc.subcore_barrier()

    @pl.loop(0, NCHUNKS)
    def _(k):
        base = e0 + k * CHUNK
        pltpu.sync_copy(src_hbm.at[pl.ds(base, CHUNK)], ids)
        pltpu.sync_copy(dst_hbm.at[pl.ds(base, CHUNK)], idc)
        pltpu.sync_copy(ones_v, dego.at[ids], add=True)
        pltpu.sync_copy(ones_v, degi.at[idc], add=True)
    plsc.subcore_barrier()

    pltpu.sync_copy(dego.at[rsl], zb)
    pltpu.sync_copy(zb, degp_hbm.at[pl.ds((cid * 2 + 0) * NPAD + r0, RPT)])
    pltpu.sync_copy(degi.at[rsl], zb)
    pltpu.sync_copy(zb, degp_hbm.at[pl.ds((cid * 2 + 1) * NPAD + r0, RPT)])


_deg_call = pl.kernel(
    _deg_body,
    out_type=jax.ShapeDtypeStruct((4 * NPAD,), jnp.float32),
    mesh=_MESH,
    compiler_params=_SC_PARAMS,
    scratch_types=[
        pltpu.VMEM_SHARED((NPAD,), jnp.float32),
        pltpu.VMEM_SHARED((NPAD,), jnp.float32),
        pltpu.VMEM((CHUNK,), jnp.int32),
        pltpu.VMEM((CHUNK,), jnp.int32),
        pltpu.VMEM((CHUNK,), jnp.float32),
        pltpu.VMEM((RPT,), jnp.float32),
    ],
)


def _tcb_body(x_ref, w1_ref, w2p_ref, b1_ref, brow_ref,
              dgo0_ref, dgo1_ref, dgi0_ref, dgi1_ref,
              g1_ref, nd_ref, nsnd_ref, crow_ref):
    hi = lax.Precision.HIGHEST
    w12 = jnp.dot(w1_ref[...], w2p_ref[...], precision=hi,
                  preferred_element_type=jnp.float32)
    y4 = jnp.dot(x_ref[...], w12, precision=hi,
                 preferred_element_type=jnp.float32) + brow_ref[...]
    dgo = dgo0_ref[...] + dgo1_ref[...]
    dgi = dgi0_ref[...] + dgi1_ref[...]
    ns = jnp.where(dgo > 0.5, lax.rsqrt(dgo), 0.0)
    nd = jnp.where(dgi > 0.5, lax.rsqrt(dgi), 0.0)
    g1_ref[...] = ns * y4
    nd_ref[...] = nd
    nsnd_ref[...] = ns * nd
    crow_ref[...] = jnp.dot(b1_ref[...], w2p_ref[...], precision=hi,
                            preferred_element_type=jnp.float32)


_BLK = 2048

_tcb_call = pl.pallas_call(
    _tcb_body,
    grid=(NPAD // _BLK,),
    in_specs=[
        pl.BlockSpec((_BLK, D_IN), lambda i: (i, 0)),
        pl.BlockSpec((D_IN, D_IN), lambda i: (0, 0)),
        pl.BlockSpec((D_IN, 8), lambda i: (0, 0)),
        pl.BlockSpec((1, D_IN), lambda i: (0, 0)),
        pl.BlockSpec((1, 8), lambda i: (0, 0)),
        pl.BlockSpec((_BLK, 1), lambda i: (i, 0)),
        pl.BlockSpec((_BLK, 1), lambda i: (i, 0)),
        pl.BlockSpec((_BLK, 1), lambda i: (i, 0)),
        pl.BlockSpec((_BLK, 1), lambda i: (i, 0)),
    ],
    out_specs=[
        pl.BlockSpec((_BLK, 8), lambda i: (i, 0)),
        pl.BlockSpec((_BLK, 1), lambda i: (i, 0)),
        pl.BlockSpec((_BLK, 1), lambda i: (i, 0)),
        pl.BlockSpec((1, 8), lambda i: (0, 0)),
    ],
    out_shape=[
        jax.ShapeDtypeStruct((NPAD, 8), jnp.float32),
        jax.ShapeDtypeStruct((NPAD, 1), jnp.float32),
        jax.ShapeDtypeStruct((NPAD, 1), jnp.float32),
        jax.ShapeDtypeStruct((1, 8), jnp.float32),
    ],
)


def _make_pass(width):
    def _pass_body(src_hbm, dst_hbm, g_hbm, z_hbm, ap0_hbm, ap1_hbm,
                   gt, at, ids, idc, mrow, rbuf):
        cid = lax.axis_index("c")
        wid = lax.axis_index("s")
        r0 = wid * RPT
        rsl = pl.ds(r0, RPT)
        e0 = (cid * NTILES + wid) * EPC

        pltpu.sync_copy(z_hbm.at[rsl], rbuf)
        pltpu.sync_copy(rbuf, at.at[rsl])
        pltpu.sync_copy(g_hbm.at[rsl], rbuf)
        pltpu.sync_copy(rbuf, gt.at[rsl])
        plsc.subcore_barrier()

        @pl.loop(0, NCHUNKS)
        def _(k):
            base = e0 + k * CHUNK
            pltpu.sync_copy(src_hbm.at[pl.ds(base, CHUNK)], ids)
            pltpu.sync_copy(dst_hbm.at[pl.ds(base, CHUNK)], idc)
            pltpu.sync_copy(gt.at[ids], mrow)
            pltpu.sync_copy(mrow, at.at[idc], add=True)
        plsc.subcore_barrier()

        pltpu.sync_copy(at.at[rsl], rbuf)

        @pl.when(cid == 0)
        def _():
            pltpu.sync_copy(rbuf, ap0_hbm.at[rsl])

        @pl.when(cid == 1)
        def _():
            pltpu.sync_copy(rbuf, ap1_hbm.at[rsl])

    return pl.kernel(
        _pass_body,
        out_type=[
            jax.ShapeDtypeStruct((NPAD, width), jnp.float32),
            jax.ShapeDtypeStruct((NPAD, width), jnp.float32),
        ],
        mesh=_MESH,
        compiler_params=_SC_PARAMS,
        scratch_types=[
            pltpu.VMEM_SHARED((NPAD, width), jnp.float32),
            pltpu.VMEM_SHARED((NPAD, width), jnp.float32),
            pltpu.VMEM((CHUNK,), jnp.int32),
            pltpu.VMEM((CHUNK,), jnp.int32),
            pltpu.VMEM((CHUNK, width), jnp.float32),
            pltpu.VMEM((RPT, width), jnp.float32),
        ],
    )


_pass1_call = _make_pass(8)
_pass2_call = _make_pass(8)


def _tcd_body(a0_ref, a1_ref, nsnd_ref, nd_ref, crow_ref, b2_ref,
              g2_ref, pc_ref):
    s1 = a0_ref[...] + a1_ref[...]
    g2_ref[...] = nsnd_ref[...] * s1
    p = nd_ref[...] * s1[:, 2:3]
    pc_ref[...] = p * crow_ref[:, :2] + b2_ref[...]


_tcd_call = pl.pallas_call(
    _tcd_body,
    out_shape=[
        jax.ShapeDtypeStruct((NPAD, 8), jnp.float32),
        jax.ShapeDtypeStruct((NPAD, 2), jnp.float32),
    ],
)


def _tcf_body(a0_ref, a1_ref, nd_ref, pc_ref, o_ref):
    s2 = a0_ref[...] + a1_ref[...]
    o_ref[...] = nd_ref[...] * s2[:, :2] + pc_ref[...]


_tcf_call = pl.pallas_call(
    _tcf_body,
    out_shape=jax.ShapeDtypeStruct((NPAD, 2), jnp.float32),
)


def kernel(feats, edge_index, W1, b1, W2, b2):
    src = edge_index[0]
    dst = edge_index[1]
    xp = jnp.zeros((NPAD, D_IN), jnp.float32).at[:N].set(feats)
    w2p = jnp.zeros((D_IN, 8), jnp.float32).at[:, :2].set(W2)
    brow = jnp.zeros((1, 8), jnp.float32).at[0, 2].set(1.0)
    b2r = b2.reshape(1, 2).astype(jnp.float32)
    z8 = jnp.zeros((NPAD, 8), jnp.float32)

    degp = _deg_call(src, dst)
    degp = degp.reshape(2, 2, NPAD, 1)
    g1, nd, nsnd, crow = _tcb_call(xp, W1, w2p, b1.reshape(1, D_IN), brow,
                                   degp[0, 0], degp[1, 0],
                                   degp[0, 1], degp[1, 1])
    a1p0, a1p1 = _pass1_call(src, dst, g1, z8)
    g2, pc = _tcd_call(a1p0, a1p1, nsnd, nd, crow, b2r)
    a2p0, a2p1 = _pass2_call(src, dst, g2, z8)
    o = _tcf_call(a2p0, a2p1, nd, pc)
    return o[:N]


# speedup vs baseline: 27.1457x; 1.3001x over previous
import jax
import jax.numpy as jnp
from jax import lax
from jax.experimental import pallas as pl
from jax.experimental.pallas import tpu as pltpu
from jax.experimental.pallas import tpu_sc as plsc

N = 10000
E = 320000
D_IN = 128
W = 16
NPAD = 10240
NCORES = 2
NTILES = 16
EPC = E // (NCORES * NTILES)
ECHUNK = 2000
RPT = NPAD // NTILES

_MESH = plsc.VectorSubcoreMesh(
    core_axis_name="c", subcore_axis_name="s", num_cores=NCORES)
_SC_PARAMS = pltpu.CompilerParams(use_tc_tiling_on_sc=False)

_IOTA = lambda: lax.iota(jnp.int32, 16)


def _rsqrt16(x):
    i = lax.bitcast_convert_type(x, jnp.int32)
    i = jnp.int32(0x5F3759DF) - lax.shift_right_logical(i, 1)
    y = lax.bitcast_convert_type(i, jnp.float32)
    for _ in range(3):
        y = y * (jnp.float32(1.5) - jnp.float32(0.5) * x * y * y)
    return jnp.where(x > jnp.float32(0.5), y, jnp.float32(0.0))


def _row_scale(rbuf, scal, n16):
    @pl.loop(0, n16)
    def _(i):
        v16 = scal[pl.ds(i * 16, 16)]
        for j in range(16):
            b = v16[jnp.full((16,), j, jnp.int32)]
            rbuf[i * 16 + j, :] = rbuf[i * 16 + j, :] * b


def _tc_body(x_ref, w1_ref, w2p_ref, b1_ref, brow_ref, y_ref, c_ref):
    hi = lax.Precision.HIGHEST
    w12 = jnp.dot(w1_ref[...], w2p_ref[...], precision=hi,
                  preferred_element_type=jnp.float32)
    y_ref[...] = jnp.dot(x_ref[...], w12, precision=hi,
                         preferred_element_type=jnp.float32) + brow_ref[...]
    c_ref[...] = jnp.dot(b1_ref[...], w2p_ref[...], precision=hi,
                         preferred_element_type=jnp.float32)


_tc_call = pl.pallas_call(
    _tc_body,
    out_shape=[
        jax.ShapeDtypeStruct((NPAD, W), jnp.float32),
        jax.ShapeDtypeStruct((1, W), jnp.float32),
    ],
)


def _deg_body(ei_hbm, degp_hbm, dego, degi, ids, idc, ones_v, zb):
    cid = lax.axis_index("c")
    wid = lax.axis_index("s")
    r0 = wid * RPT
    rsl = pl.ds(r0, RPT)
    e0 = (cid * NTILES + wid) * EPC

    zero16 = jnp.zeros((16,), jnp.float32)
    one16 = jnp.full((16,), 1.0, jnp.float32)

    @pl.loop(0, RPT // 16)
    def _(i):
        zb[pl.ds(i * 16, 16)] = zero16

    @pl.loop(0, EPC // 16)
    def _(i):
        ones_v[pl.ds(i * 16, 16)] = one16

    pltpu.sync_copy(zb, dego.at[rsl])
    pltpu.sync_copy(zb, degi.at[rsl])
    plsc.subcore_barrier()

    e0a = pl.multiple_of(e0, 8)
    pltpu.sync_copy(ei_hbm.at[0, pl.ds(e0a, EPC)], ids)
    pltpu.sync_copy(ei_hbm.at[1, pl.ds(e0a, EPC)], idc)
    pltpu.sync_copy(ones_v, dego.at[ids], add=True)
    pltpu.sync_copy(ones_v, degi.at[idc], add=True)
    plsc.subcore_barrier()

    pltpu.sync_copy(dego.at[rsl], zb)
    pltpu.sync_copy(zb, degp_hbm.at[pl.ds((cid * 2 + 0) * NPAD + r0, RPT)])
    pltpu.sync_copy(degi.at[rsl], zb)
    pltpu.sync_copy(zb, degp_hbm.at[pl.ds((cid * 2 + 1) * NPAD + r0, RPT)])


_deg_call = pl.kernel(
    _deg_body,
    out_type=jax.ShapeDtypeStruct((4 * NPAD,), jnp.float32),
    mesh=_MESH,
    compiler_params=_SC_PARAMS,
    scratch_types=[
        pltpu.VMEM_SHARED((NPAD,), jnp.float32),
        pltpu.VMEM_SHARED((NPAD,), jnp.float32),
        pltpu.VMEM((EPC,), jnp.int32),
        pltpu.VMEM((EPC,), jnp.int32),
        pltpu.VMEM((EPC,), jnp.float32),
        pltpu.VMEM((RPT,), jnp.float32),
    ],
)


def _edge_pass(ei_hbm, gt, at, ids, idc, mrow, e0):
    @pl.loop(0, EPC // ECHUNK)
    def _(k):
        base = pl.multiple_of(e0 + k * ECHUNK, 8)
        pltpu.sync_copy(ei_hbm.at[0, pl.ds(base, ECHUNK)], ids)
        pltpu.sync_copy(ei_hbm.at[1, pl.ds(base, ECHUNK)], idc)
        pltpu.sync_copy(gt.at[ids], mrow)
        pltpu.sync_copy(mrow, at.at[idc], add=True)


def _zero_rows(rbuf):
    zero16 = jnp.zeros((16,), jnp.float32)

    @pl.loop(0, RPT)
    def _(r):
        rbuf[r, :] = zero16


def _merged_deg(degp_hbm, which, r0, wa, wb, nrows=RPT):
    pltpu.sync_copy(degp_hbm.at[pl.ds(which * NPAD + r0, nrows)], wa)
    pltpu.sync_copy(degp_hbm.at[pl.ds((2 + which) * NPAD + r0, nrows)], wb)

    @pl.loop(0, nrows // 16)
    def _(i):
        sl = pl.ds(i * 16, 16)
        wa[sl] = wa[sl] + wb[sl]


def _p1_body(ei_hbm, y_hbm, degp_hbm, pa0_hbm, pa1_hbm,
             gt, at, ids, idc, mrow, rbuf, nsb, wb):
    cid = lax.axis_index("c")
    wid = lax.axis_index("s")
    r0 = wid * RPT
    rsl = pl.ds(r0, RPT)
    e0 = (cid * NTILES + wid) * EPC

    _merged_deg(degp_hbm, 0, r0, nsb, wb)

    @pl.loop(0, RPT // 16)
    def _(i):
        sl = pl.ds(i * 16, 16)
        nsb[sl] = _rsqrt16(nsb[sl])

    pltpu.sync_copy(y_hbm.at[rsl], rbuf)
    _row_scale(rbuf, nsb, RPT // 16)
    pltpu.sync_copy(rbuf, gt.at[rsl])
    _zero_rows(rbuf)
    pltpu.sync_copy(rbuf, at.at[rsl])
    plsc.subcore_barrier()

    _edge_pass(ei_hbm, gt, at, ids, idc, mrow, e0)
    plsc.subcore_barrier()

    pltpu.sync_copy(at.at[rsl], rbuf)

    @pl.when(cid == 0)
    def _():
        pltpu.sync_copy(rbuf, pa0_hbm.at[rsl])

    @pl.when(cid == 1)
    def _():
        pltpu.sync_copy(rbuf, pa1_hbm.at[rsl])


def _p2_body(ei_hbm, pa0_hbm, pa1_hbm, degp_hbm, sa0_hbm, sa1_hbm, pr_hbm,
             gt, at, ids, idc, mrow, rbuf, rbuf2, nsb, ndb, wb):
    cid = lax.axis_index("c")
    wid = lax.axis_index("s")
    r0 = wid * RPT
    rsl = pl.ds(r0, RPT)
    e0 = (cid * NTILES + wid) * EPC

    _merged_deg(degp_hbm, 0, r0, nsb, wb)
    _merged_deg(degp_hbm, 1, r0, ndb, wb)

    @pl.loop(0, RPT // 16)
    def _(i):
        sl = pl.ds(i * 16, 16)
        nsb[sl] = _rsqrt16(nsb[sl])
        ndb[sl] = _rsqrt16(ndb[sl])
        nsb[sl] = nsb[sl] * ndb[sl]

    pltpu.sync_copy(pa0_hbm.at[rsl], rbuf)
    pltpu.sync_copy(pa1_hbm.at[rsl], rbuf2)

    @pl.loop(0, RPT)
    def _(r):
        rbuf[r, :] = rbuf[r, :] + rbuf2[r, :]
        rbuf2[r, :] = rbuf[r, :]

    _row_scale(rbuf2, ndb, RPT // 16)

    @pl.when(cid == 0)
    def _():
        pltpu.sync_copy(rbuf2, pr_hbm.at[rsl])

    _row_scale(rbuf, nsb, RPT // 16)
    pltpu.sync_copy(rbuf, gt.at[rsl])
    _zero_rows(rbuf)
    pltpu.sync_copy(rbuf, at.at[rsl])
    plsc.subcore_barrier()

    _edge_pass(ei_hbm, gt, at, ids, idc, mrow, e0)
    plsc.subcore_barrier()

    pltpu.sync_copy(at.at[rsl], rbuf)

    @pl.when(cid == 0)
    def _():
        pltpu.sync_copy(rbuf, sa0_hbm.at[rsl])

    @pl.when(cid == 1)
    def _():
        pltpu.sync_copy(rbuf, sa1_hbm.at[rsl])


def _epi_body(sa0_hbm, sa1_hbm, pr_hbm, degp_hbm, cv_hbm, o_hbm,
              rbuf, rbuf2, prb, ndb, wb, cb):
    cid = lax.axis_index("c")
    wid = lax.axis_index("s")
    half = RPT // 2
    r0 = (cid * NTILES + wid) * half
    rsl = pl.ds(r0, half)

    _merged_deg(degp_hbm, 1, r0, ndb, wb, nrows=half)

    @pl.loop(0, half // 16)
    def _(i):
        sl = pl.ds(i * 16, 16)
        ndb[sl] = _rsqrt16(ndb[sl])

    pltpu.sync_copy(sa0_hbm.at[rsl], rbuf)
    pltpu.sync_copy(sa1_hbm.at[rsl], rbuf2)
    pltpu.sync_copy(pr_hbm.at[rsl], prb)
    pltpu.sync_copy(cv_hbm, cb)
    it = _IOTA()
    cvv = cb[pl.ds(0, 16)]
    cv01 = jnp.where(it < 2, cvv, jnp.float32(0.0))
    b2v = cvv[(it + 2) & 15]
    b2v01 = jnp.where(it < 2, b2v, jnp.float32(0.0))

    @pl.loop(0, half)
    def _(r):
        rbuf[r, :] = rbuf[r, :] + rbuf2[r, :]

    _row_scale(rbuf, ndb, half // 16)

    @pl.loop(0, half)
    def _(r):
        pv = prb[r, :]
        p = pv[jnp.full((16,), 2, jnp.int32)]
        rbuf[r, :] = rbuf[r, :] + p * cv01 + b2v01

    pltpu.sync_copy(rbuf, o_hbm.at[rsl])


def _mk(body, n_out, w_out, scratch):
    outs = [jax.ShapeDtypeStruct((NPAD, w), jnp.float32) for w in w_out]
    return pl.kernel(
        body,
        out_type=outs if n_out > 1 else outs[0],
        mesh=_MESH,
        compiler_params=_SC_PARAMS,
        scratch_types=scratch,
    )


_p1_call = _mk(_p1_body, 2, [W, W], [
    pltpu.VMEM_SHARED((NPAD, W), jnp.float32),
    pltpu.VMEM_SHARED((NPAD, W), jnp.float32),
    pltpu.VMEM((ECHUNK,), jnp.int32),
    pltpu.VMEM((ECHUNK,), jnp.int32),
    pltpu.VMEM((ECHUNK, W), jnp.float32),
    pltpu.VMEM((RPT, W), jnp.float32),
    pltpu.VMEM((RPT,), jnp.float32),
    pltpu.VMEM((RPT,), jnp.float32),
])

_p2_call = _mk(_p2_body, 3, [W, W, W], [
    pltpu.VMEM_SHARED((NPAD, W), jnp.float32),
    pltpu.VMEM_SHARED((NPAD, W), jnp.float32),
    pltpu.VMEM((ECHUNK,), jnp.int32),
    pltpu.VMEM((ECHUNK,), jnp.int32),
    pltpu.VMEM((ECHUNK, W), jnp.float32),
    pltpu.VMEM((RPT, W), jnp.float32),
    pltpu.VMEM((RPT, W), jnp.float32),
    pltpu.VMEM((RPT,), jnp.float32),
    pltpu.VMEM((RPT,), jnp.float32),
    pltpu.VMEM((RPT,), jnp.float32),
])

_epi_call = _mk(_epi_body, 1, [W], [
    pltpu.VMEM((RPT // 2, W), jnp.float32),
    pltpu.VMEM((RPT // 2, W), jnp.float32),
    pltpu.VMEM((RPT // 2, W), jnp.float32),
    pltpu.VMEM((RPT // 2,), jnp.float32),
    pltpu.VMEM((RPT // 2,), jnp.float32),
    pltpu.VMEM((16,), jnp.float32),
])


def kernel(feats, edge_index, W1, b1, W2, b2):
    xp = jnp.zeros((NPAD, D_IN), jnp.float32).at[:N].set(feats)
    w2p = jnp.zeros((D_IN, W), jnp.float32).at[:, :2].set(W2)
    brow = jnp.zeros((1, W), jnp.float32).at[0, 2].set(1.0)

    y16, crow = _tc_call(xp, W1, w2p, b1.reshape(1, D_IN), brow)
    cvec = jnp.concatenate(
        [crow[0, :2], b2.astype(jnp.float32), jnp.zeros((12,), jnp.float32)])

    degp = _deg_call(edge_index)
    pa0, pa1 = _p1_call(edge_index, y16, degp)
    sa0, sa1, pr = _p2_call(edge_index, pa0, pa1, degp)
    o16 = _epi_call(sa0, sa1, pr, degp, cvec)
    return o16[:N, :2]
